# baseline (speedup 1.0000x reference)
import functools
import jax
import jax.numpy as jnp
from jax.experimental import pallas as pl
from jax.experimental.pallas import tpu as pltpu


def _pick_block(total, cap):
    best = 1
    for b in range(1, cap + 1):
        if total % b == 0:
            best = b
    return best


def _mm_kernel(x_ref, wl_ref, wr_ref, xl_ref, xr_ref):
    xl_ref[0] = jnp.dot(x_ref[...], wl_ref[0],
                        preferred_element_type=jnp.float32)
    xr_ref[0] = jnp.dot(x_ref[...], wr_ref[0],
                        preferred_element_type=jnp.float32)


def _edge_kernel(xl_hbm, xr_hbm, aa_ref, edges_ref, agg_hbm,
                 xl_s, xr_s, aggu_s, den_s, sem, *, eb_count, eblk):
    h = pl.program_id(0)
    eb = pl.program_id(1)

    @pl.when(eb == 0)
    def _init():
        cp = pltpu.make_async_copy(xl_hbm.at[h], xl_s, sem)
        cp.start()
        cp.wait()
        cp = pltpu.make_async_copy(xr_hbm.at[h], xr_s, sem)
        cp.start()
        cp.wait()
        aggu_s[...] = jnp.zeros_like(aggu_s)
        den_s[...] = jnp.zeros_like(den_s)

    aa = aa_ref[0]

    def body(i, carry):
        src = edges_ref[0, 0, i]
        dst = edges_ref[0, 1, i]
        xlrow = xl_s[pl.ds(src, 1), :]
        xrrow = xr_s[pl.ds(dst, 1), :]
        m = xlrow + xrrow
        e = jnp.where(m > 0, m, 0.2 * m)
        logit = jnp.sum(e * aa, axis=1, keepdims=True)
        ev = jnp.exp(logit)
        aggu_s[pl.ds(dst, 1), :] += ev * xlrow
        den_s[pl.ds(dst, 1), :] += ev
        return carry

    jax.lax.fori_loop(0, eblk, body, 0)

    @pl.when(eb == eb_count - 1)
    def _fin():
        aggu_s[...] = aggu_s[...] / (den_s[:, 0:1] + 1e-16)
        cp = pltpu.make_async_copy(aggu_s, agg_hbm.at[h], sem)
        cp.start()
        cp.wait()


def _head_kernel(agg_ref, gb_ref, av_ref, ab_ref, cw_ref, cb_ref, out_ref,
                 *, nsub):
    z = jnp.mean(agg_ref[...], axis=0) + gb_ref[...]
    z = jnp.maximum(z, 0.0)
    parts = [z[:, k * nsub:(k + 1) * nsub] for k in range(3)]
    scs = []
    for k in range(3):
        s = jnp.sum(parts[k] * av_ref[k:k + 1, :], axis=1, keepdims=True)
        s = s + ab_ref[k, 0]
        scs.append(jnp.where(s > 0, s, 0.01 * s))
    scores = jnp.concatenate(scs, axis=1)
    mx = jnp.max(scores, axis=1, keepdims=True)
    ex = jnp.exp(scores - mx)
    alpha = ex / jnp.sum(ex, axis=1, keepdims=True)
    zw = jnp.concatenate([parts[i] * alpha[:, i:i + 1] for i in range(3)],
                         axis=1)
    out_ref[...] = jnp.dot(zw, cw_ref[...],
                           preferred_element_type=jnp.float32) + cb_ref[...]


def kernel(x, edge_index, W_l, W_r, att_a, gat_bias, att_vec, att_bias,
           cls_W, cls_b):
    N, D = x.shape
    E = edge_index.shape[1]
    H, C = att_a.shape
    nsub = att_vec.shape[1]
    outd = cls_W.shape[1]

    eblk = _pick_block(E, 2048)
    ebc = E // eblk
    edges_r = edge_index.reshape(2, ebc, eblk).transpose(1, 0, 2)
    wl = W_l.reshape(D, H, C).transpose(1, 0, 2)
    wr = W_r.reshape(D, H, C).transpose(1, 0, 2)
    aa = att_a.reshape(H, 1, C)

    nblk = 1
    for b in range(8, 2049, 8):
        if N % b == 0:
            nblk = b

    xl_all, xr_all = pl.pallas_call(
        _mm_kernel,
        grid=(H, N // nblk),
        in_specs=[
            pl.BlockSpec((nblk, D), lambda h, i: (i, 0)),
            pl.BlockSpec((1, D, C), lambda h, i: (h, 0, 0)),
            pl.BlockSpec((1, D, C), lambda h, i: (h, 0, 0)),
        ],
        out_specs=[
            pl.BlockSpec((1, nblk, C), lambda h, i: (h, i, 0)),
            pl.BlockSpec((1, nblk, C), lambda h, i: (h, i, 0)),
        ],
        out_shape=[
            jax.ShapeDtypeStruct((H, N, C), jnp.float32),
            jax.ShapeDtypeStruct((H, N, C), jnp.float32),
        ],
    )(x, wl, wr)

    agg = pl.pallas_call(
        functools.partial(_edge_kernel, eb_count=ebc, eblk=eblk),
        grid=(H, ebc),
        in_specs=[
            pl.BlockSpec(memory_space=pl.ANY),
            pl.BlockSpec(memory_space=pl.ANY),
            pl.BlockSpec((1, 1, C), lambda h, eb: (h, 0, 0)),
            pl.BlockSpec((1, 2, eblk), lambda h, eb: (eb, 0, 0),
                         memory_space=pltpu.SMEM),
        ],
        out_specs=pl.BlockSpec(memory_space=pl.ANY),
        out_shape=jax.ShapeDtypeStruct((H, N, C), jnp.float32),
        scratch_shapes=[
            pltpu.VMEM((N, C), jnp.float32),
            pltpu.VMEM((N, C), jnp.float32),
            pltpu.VMEM((N, C), jnp.float32),
            pltpu.VMEM((N, 8), jnp.float32),
            pltpu.SemaphoreType.DMA,
        ],
    )(xl_all, xr_all, aa, edges_r)

    out = pl.pallas_call(
        functools.partial(_head_kernel, nsub=nsub),
        grid=(N // nblk,),
        in_specs=[
            pl.BlockSpec((H, nblk, C), lambda i: (0, i, 0)),
            pl.BlockSpec((1, C), lambda i: (0, 0)),
            pl.BlockSpec((3, nsub), lambda i: (0, 0)),
            pl.BlockSpec((3, 1), lambda i: (0, 0), memory_space=pltpu.SMEM),
            pl.BlockSpec((C, outd), lambda i: (0, 0)),
            pl.BlockSpec((1, outd), lambda i: (0, 0)),
        ],
        out_specs=pl.BlockSpec((nblk, outd), lambda i: (i, 0)),
        out_shape=jax.ShapeDtypeStruct((N, outd), jnp.float32),
    )(agg, gat_bias.reshape(1, C), att_vec, att_bias, cls_W,
      cls_b.reshape(1, outd))
    return out


# speedup vs baseline: 5.7610x; 5.7610x over previous
import functools
import jax
import jax.numpy as jnp
from jax.experimental import pallas as pl
from jax.experimental.pallas import tpu as pltpu


def _pick_block(total, cap):
    best = 1
    for b in range(1, cap + 1):
        if total % b == 0:
            best = b
    return best


def _mm_kernel(x_ref, wl_ref, wr_ref, xl_ref, xr_ref):
    xl_ref[0] = jnp.dot(x_ref[...], wl_ref[0],
                        preferred_element_type=jnp.float32)
    xr_ref[0] = jnp.dot(x_ref[...], wr_ref[0],
                        preferred_element_type=jnp.float32)


def _edge_kernel(xl_hbm, xr_hbm, aa_ref, edges_ref, agg_hbm,
                 xl_s, xr_s, agg2_s, sem, *, eb_count, eblk, unroll, nc):
    h = pl.program_id(0)
    eb = pl.program_id(1)

    @pl.when(eb == 0)
    def _init():
        cp = pltpu.make_async_copy(xl_hbm.at[h], xl_s, sem)
        cp.start()
        cp.wait()
        cp = pltpu.make_async_copy(xr_hbm.at[h], xr_s, sem)
        cp.start()
        cp.wait()
        agg2_s[...] = jnp.zeros_like(agg2_s)

    aa = aa_ref[0]

    def body(ib, carry):
        base = ib * unroll
        msgs = []
        dsts = []
        for j in range(unroll):
            src = edges_ref[0, 0, base + j]
            dst = edges_ref[0, 1, base + j]
            xlrow = xl_s[pl.ds(src, 1), :]
            xrrow = xr_s[pl.ds(dst, 1), :]
            m = xlrow + xrrow
            e = jnp.where(m > 0, m, 0.2 * m)
            logit = jnp.sum(e * aa, axis=1, keepdims=True)
            ev = jnp.exp(logit)
            msgs.append(jnp.concatenate(
                [ev * xlrow, jnp.broadcast_to(ev, (1, 128))], axis=1))
            dsts.append(dst)
        for j in range(unroll):
            agg2_s[pl.ds(dsts[j], 1), :] += msgs[j]
        return carry

    jax.lax.fori_loop(0, eblk // unroll, body, 0)

    @pl.when(eb == eb_count - 1)
    def _fin():
        xl_s[...] = agg2_s[:, :nc] / (agg2_s[:, nc:nc + 1] + 1e-16)
        cp = pltpu.make_async_copy(xl_s, agg_hbm.at[h], sem)
        cp.start()
        cp.wait()


def _head_kernel(agg_ref, gb_ref, av_ref, ab_ref, cw_ref, cb_ref, out_ref,
                 *, nsub):
    z = jnp.mean(agg_ref[...], axis=0) + gb_ref[...]
    z = jnp.maximum(z, 0.0)
    parts = [z[:, k * nsub:(k + 1) * nsub] for k in range(3)]
    scs = []
    for k in range(3):
        s = jnp.sum(parts[k] * av_ref[k:k + 1, :], axis=1, keepdims=True)
        s = s + ab_ref[k, 0]
        scs.append(jnp.where(s > 0, s, 0.01 * s))
    scores = jnp.concatenate(scs, axis=1)
    mx = jnp.max(scores, axis=1, keepdims=True)
    ex = jnp.exp(scores - mx)
    alpha = ex / jnp.sum(ex, axis=1, keepdims=True)
    zw = jnp.concatenate([parts[i] * alpha[:, i:i + 1] for i in range(3)],
                         axis=1)
    out_ref[...] = jnp.dot(zw, cw_ref[...],
                           preferred_element_type=jnp.float32) + cb_ref[...]


def kernel(x, edge_index, W_l, W_r, att_a, gat_bias, att_vec, att_bias,
           cls_W, cls_b):
    N, D = x.shape
    E = edge_index.shape[1]
    H, C = att_a.shape
    nsub = att_vec.shape[1]
    outd = cls_W.shape[1]

    eblk = _pick_block(E, 2048)
    ebc = E // eblk
    edges_r = edge_index.reshape(2, ebc, eblk).transpose(1, 0, 2)
    wl = W_l.reshape(D, H, C).transpose(1, 0, 2)
    wr = W_r.reshape(D, H, C).transpose(1, 0, 2)
    aa = att_a.reshape(H, 1, C)

    nblk = 1
    for b in range(8, 2049, 8):
        if N % b == 0:
            nblk = b

    xl_all, xr_all = pl.pallas_call(
        _mm_kernel,
        grid=(H, N // nblk),
        in_specs=[
            pl.BlockSpec((nblk, D), lambda h, i: (i, 0)),
            pl.BlockSpec((1, D, C), lambda h, i: (h, 0, 0)),
            pl.BlockSpec((1, D, C), lambda h, i: (h, 0, 0)),
        ],
        out_specs=[
            pl.BlockSpec((1, nblk, C), lambda h, i: (h, i, 0)),
            pl.BlockSpec((1, nblk, C), lambda h, i: (h, i, 0)),
        ],
        out_shape=[
            jax.ShapeDtypeStruct((H, N, C), jnp.float32),
            jax.ShapeDtypeStruct((H, N, C), jnp.float32),
        ],
    )(x, wl, wr)

    agg = pl.pallas_call(
        functools.partial(_edge_kernel, eb_count=ebc, eblk=eblk,
                          unroll=max(u for u in (1, 2, 4, 8)
                                     if eblk % u == 0), nc=C),
        grid=(H, ebc),
        in_specs=[
            pl.BlockSpec(memory_space=pl.ANY),
            pl.BlockSpec(memory_space=pl.ANY),
            pl.BlockSpec((1, 1, C), lambda h, eb: (h, 0, 0)),
            pl.BlockSpec((1, 2, eblk), lambda h, eb: (eb, 0, 0),
                         memory_space=pltpu.SMEM),
        ],
        out_specs=pl.BlockSpec(memory_space=pl.ANY),
        out_shape=jax.ShapeDtypeStruct((H, N, C), jnp.float32),
        scratch_shapes=[
            pltpu.VMEM((N, C), jnp.float32),
            pltpu.VMEM((N, C), jnp.float32),
            pltpu.VMEM((N, C + 128), jnp.float32),
            pltpu.SemaphoreType.DMA,
        ],
    )(xl_all, xr_all, aa, edges_r)

    out = pl.pallas_call(
        functools.partial(_head_kernel, nsub=nsub),
        grid=(N // nblk,),
        in_specs=[
            pl.BlockSpec((H, nblk, C), lambda i: (0, i, 0)),
            pl.BlockSpec((1, C), lambda i: (0, 0)),
            pl.BlockSpec((3, nsub), lambda i: (0, 0)),
            pl.BlockSpec((3, 1), lambda i: (0, 0), memory_space=pltpu.SMEM),
            pl.BlockSpec((C, outd), lambda i: (0, 0)),
            pl.BlockSpec((1, outd), lambda i: (0, 0)),
        ],
        out_specs=pl.BlockSpec((nblk, outd), lambda i: (i, 0)),
        out_shape=jax.ShapeDtypeStruct((N, outd), jnp.float32),
    )(agg, gat_bias.reshape(1, C), att_vec, att_bias, cls_W,
      cls_b.reshape(1, outd))
    return out


# speedup vs baseline: 8.3479x; 1.4490x over previous
import functools
import jax
import jax.numpy as jnp
from jax.experimental import pallas as pl
from jax.experimental.pallas import tpu as pltpu


def _pick_block(total, cap):
    best = 1
    for b in range(1, cap + 1):
        if total % b == 0:
            best = b
    return best


def _mm_kernel(x_ref, wl_ref, wr_ref, xl_ref, xr_ref):
    xl_ref[0] = jnp.dot(x_ref[...], wl_ref[0],
                        preferred_element_type=jnp.float32)
    xr_ref[0] = jnp.dot(x_ref[...], wr_ref[0],
                        preferred_element_type=jnp.float32)


def _edge_kernel(xl_hbm, xr_hbm, aa_ref, edges_ref, agg_hbm,
                 xl_s, xr_s, agg2_s, sem, *, eb_count, eblk, unroll, nc):
    h = pl.program_id(0)
    eb = pl.program_id(1)

    @pl.when(eb == 0)
    def _init():
        cp = pltpu.make_async_copy(xl_hbm.at[h], xl_s, sem)
        cp.start()
        cp.wait()
        cp = pltpu.make_async_copy(xr_hbm.at[h], xr_s, sem)
        cp.start()
        cp.wait()
        agg2_s[...] = jnp.zeros_like(agg2_s)

    aa = aa_ref[0]

    def body(ib, carry):
        base = ib * unroll
        msgs = []
        dsts = []
        for j in range(unroll):
            src = edges_ref[0, 0, base + j]
            dst = edges_ref[0, 1, base + j]
            xlrow = xl_s[pl.ds(src, 1), :]
            xrrow = xr_s[pl.ds(dst, 1), :]
            m = xlrow + xrrow
            e = jnp.where(m > 0, m, 0.2 * m)
            logit = jnp.sum(e * aa, axis=1, keepdims=True)
            ev = jnp.exp(logit)
            msgs.append(jnp.concatenate(
                [ev * xlrow, jnp.broadcast_to(ev, (1, 128))], axis=1))
            dsts.append(dst)
        for j in range(unroll):
            agg2_s[pl.ds(dsts[j], 1), :] += msgs[j]
        return carry

    jax.lax.fori_loop(0, eblk // unroll, body, 0)

    @pl.when(eb == eb_count - 1)
    def _fin():
        xl_s[...] = agg2_s[:, :nc] / (agg2_s[:, nc:nc + 1] + 1e-16)
        cp = pltpu.make_async_copy(xl_s, agg_hbm.at[h], sem)
        cp.start()
        cp.wait()


def _head_kernel(agg_ref, gb_ref, av_ref, ab_ref, cw_ref, cb_ref, out_ref,
                 *, nsub):
    z = jnp.mean(agg_ref[...], axis=0) + gb_ref[...]
    z = jnp.maximum(z, 0.0)
    parts = [z[:, k * nsub:(k + 1) * nsub] for k in range(3)]
    scs = []
    for k in range(3):
        s = jnp.sum(parts[k] * av_ref[k:k + 1, :], axis=1, keepdims=True)
        s = s + ab_ref[k, 0]
        scs.append(jnp.where(s > 0, s, 0.01 * s))
    scores = jnp.concatenate(scs, axis=1)
    mx = jnp.max(scores, axis=1, keepdims=True)
    ex = jnp.exp(scores - mx)
    alpha = ex / jnp.sum(ex, axis=1, keepdims=True)
    zw = jnp.concatenate([parts[i] * alpha[:, i:i + 1] for i in range(3)],
                         axis=1)
    out_ref[...] = jnp.dot(zw, cw_ref[...],
                           preferred_element_type=jnp.float32) + cb_ref[...]


def kernel(x, edge_index, W_l, W_r, att_a, gat_bias, att_vec, att_bias,
           cls_W, cls_b):
    N, D = x.shape
    E = edge_index.shape[1]
    H, C = att_a.shape
    nsub = att_vec.shape[1]
    outd = cls_W.shape[1]

    eblk = _pick_block(E, 2048)
    ebc = E // eblk
    edges_r = edge_index.reshape(2, ebc, eblk).transpose(1, 0, 2)
    wl = W_l.reshape(D, H, C).transpose(1, 0, 2)
    wr = W_r.reshape(D, H, C).transpose(1, 0, 2)
    aa = att_a.reshape(H, 1, C)

    nblk = 1
    for b in range(8, 2049, 8):
        if N % b == 0:
            nblk = b

    xl_all, xr_all = pl.pallas_call(
        _mm_kernel,
        grid=(H, N // nblk),
        in_specs=[
            pl.BlockSpec((nblk, D), lambda h, i: (i, 0)),
            pl.BlockSpec((1, D, C), lambda h, i: (h, 0, 0)),
            pl.BlockSpec((1, D, C), lambda h, i: (h, 0, 0)),
        ],
        out_specs=[
            pl.BlockSpec((1, nblk, C), lambda h, i: (h, i, 0)),
            pl.BlockSpec((1, nblk, C), lambda h, i: (h, i, 0)),
        ],
        out_shape=[
            jax.ShapeDtypeStruct((H, N, C), jnp.float32),
            jax.ShapeDtypeStruct((H, N, C), jnp.float32),
        ],
    )(x, wl, wr)

    agg = pl.pallas_call(
        functools.partial(_edge_kernel, eb_count=ebc, eblk=eblk,
                          unroll=max(u for u in (1, 2, 4, 8, 16)
                                     if eblk % u == 0), nc=C),
        grid=(H, ebc),
        in_specs=[
            pl.BlockSpec(memory_space=pl.ANY),
            pl.BlockSpec(memory_space=pl.ANY),
            pl.BlockSpec((1, 1, C), lambda h, eb: (h, 0, 0)),
            pl.BlockSpec((1, 2, eblk), lambda h, eb: (eb, 0, 0),
                         memory_space=pltpu.SMEM),
        ],
        out_specs=pl.BlockSpec(memory_space=pl.ANY),
        out_shape=jax.ShapeDtypeStruct((H, N, C), jnp.float32),
        scratch_shapes=[
            pltpu.VMEM((N, C), jnp.float32),
            pltpu.VMEM((N, C), jnp.float32),
            pltpu.VMEM((N, C + 128), jnp.float32),
            pltpu.SemaphoreType.DMA,
        ],
    )(xl_all, xr_all, aa, edges_r)

    out = pl.pallas_call(
        functools.partial(_head_kernel, nsub=nsub),
        grid=(N // nblk,),
        in_specs=[
            pl.BlockSpec((H, nblk, C), lambda i: (0, i, 0)),
            pl.BlockSpec((1, C), lambda i: (0, 0)),
            pl.BlockSpec((3, nsub), lambda i: (0, 0)),
            pl.BlockSpec((3, 1), lambda i: (0, 0), memory_space=pltpu.SMEM),
            pl.BlockSpec((C, outd), lambda i: (0, 0)),
            pl.BlockSpec((1, outd), lambda i: (0, 0)),
        ],
        out_specs=pl.BlockSpec((nblk, outd), lambda i: (i, 0)),
        out_shape=jax.ShapeDtypeStruct((N, outd), jnp.float32),
    )(agg, gat_bias.reshape(1, C), att_vec, att_bias, cls_W,
      cls_b.reshape(1, outd))
    return out


# speedup vs baseline: 10.4283x; 1.2492x over previous
import functools
import jax
import jax.numpy as jnp
from jax.experimental import pallas as pl
from jax.experimental.pallas import tpu as pltpu


def _pick_block(total, cap):
    best = 1
    for b in range(1, cap + 1):
        if total % b == 0:
            best = b
    return best


def _mm_kernel(x_ref, wl_ref, wr_ref, xl_ref, xr_ref):
    xl_ref[0] = jnp.dot(x_ref[...], wl_ref[0],
                        preferred_element_type=jnp.float32)
    xr_ref[0] = jnp.dot(x_ref[...], wr_ref[0],
                        preferred_element_type=jnp.float32)


def _edge_kernel(xl_hbm, xr_hbm, aa_ref, edges_ref, agg_hbm,
                 xl_s, xr_s, agg2_s, sem, *, eb_count, eblk, unroll, nc):
    h = pl.program_id(0)
    eb = pl.program_id(1)

    @pl.when(eb == 0)
    def _init():
        cp = pltpu.make_async_copy(xl_hbm.at[h], xl_s, sem)
        cp.start()
        cp.wait()
        cp = pltpu.make_async_copy(xr_hbm.at[h], xr_s, sem)
        cp.start()
        cp.wait()
        agg2_s[...] = jnp.zeros_like(agg2_s)

    aa = aa_ref[0]

    def body(ib, carry):
        base = ib * unroll
        msgs = []
        dsts = []
        for j in range(unroll):
            src = edges_ref[0, 0, base + j]
            dst = edges_ref[0, 1, base + j]
            xlrow = xl_s[pl.ds(src, 1), :]
            xrrow = xr_s[pl.ds(dst, 1), :]
            m = xlrow + xrrow
            e = jnp.where(m > 0, m, 0.2 * m)
            logit = jnp.sum(e * aa, axis=1, keepdims=True)
            ev = jnp.exp(logit)
            msgs.append(jnp.concatenate(
                [ev * xlrow, jnp.broadcast_to(ev, (1, 128))], axis=1))
            dsts.append(dst)
        for j in range(unroll):
            agg2_s[pl.ds(dsts[j], 1), :] += msgs[j]
        return carry

    jax.lax.fori_loop(0, eblk // unroll, body, 0)

    @pl.when(eb == eb_count - 1)
    def _fin():
        xl_s[...] = agg2_s[:, :nc] / (agg2_s[:, nc:nc + 1] + 1e-16)
        cp = pltpu.make_async_copy(xl_s, agg_hbm.at[h], sem)
        cp.start()
        cp.wait()


def _head_kernel(agg_ref, gb_ref, av_ref, ab_ref, cw_ref, cb_ref, out_ref,
                 *, nsub):
    z = jnp.mean(agg_ref[...], axis=0) + gb_ref[...]
    z = jnp.maximum(z, 0.0)
    parts = [z[:, k * nsub:(k + 1) * nsub] for k in range(3)]
    scs = []
    for k in range(3):
        s = jnp.sum(parts[k] * av_ref[k:k + 1, :], axis=1, keepdims=True)
        s = s + ab_ref[k, 0]
        scs.append(jnp.where(s > 0, s, 0.01 * s))
    scores = jnp.concatenate(scs, axis=1)
    mx = jnp.max(scores, axis=1, keepdims=True)
    ex = jnp.exp(scores - mx)
    alpha = ex / jnp.sum(ex, axis=1, keepdims=True)
    zw = jnp.concatenate([parts[i] * alpha[:, i:i + 1] for i in range(3)],
                         axis=1)
    out_ref[...] = jnp.dot(zw, cw_ref[...],
                           preferred_element_type=jnp.float32) + cb_ref[...]


def kernel(x, edge_index, W_l, W_r, att_a, gat_bias, att_vec, att_bias,
           cls_W, cls_b):
    N, D = x.shape
    E = edge_index.shape[1]
    H, C = att_a.shape
    nsub = att_vec.shape[1]
    outd = cls_W.shape[1]

    eblk = _pick_block(E, 2048)
    for b in range(32, 2049, 32):
        if E % b == 0:
            eblk = b
    ebc = E // eblk
    edges_r = edge_index.reshape(2, ebc, eblk).transpose(1, 0, 2)
    wl = W_l.reshape(D, H, C).transpose(1, 0, 2)
    wr = W_r.reshape(D, H, C).transpose(1, 0, 2)
    aa = att_a.reshape(H, 1, C)

    nblk = 1
    for b in range(8, 2049, 8):
        if N % b == 0:
            nblk = b

    xl_all, xr_all = pl.pallas_call(
        _mm_kernel,
        grid=(H, N // nblk),
        in_specs=[
            pl.BlockSpec((nblk, D), lambda h, i: (i, 0)),
            pl.BlockSpec((1, D, C), lambda h, i: (h, 0, 0)),
            pl.BlockSpec((1, D, C), lambda h, i: (h, 0, 0)),
        ],
        out_specs=[
            pl.BlockSpec((1, nblk, C), lambda h, i: (h, i, 0)),
            pl.BlockSpec((1, nblk, C), lambda h, i: (h, i, 0)),
        ],
        out_shape=[
            jax.ShapeDtypeStruct((H, N, C), jnp.float32),
            jax.ShapeDtypeStruct((H, N, C), jnp.float32),
        ],
    )(x, wl, wr)

    agg = pl.pallas_call(
        functools.partial(_edge_kernel, eb_count=ebc, eblk=eblk,
                          unroll=max(u for u in (1, 2, 4, 8, 16, 32)
                                     if eblk % u == 0), nc=C),
        grid=(H, ebc),
        in_specs=[
            pl.BlockSpec(memory_space=pl.ANY),
            pl.BlockSpec(memory_space=pl.ANY),
            pl.BlockSpec((1, 1, C), lambda h, eb: (h, 0, 0)),
            pl.BlockSpec((1, 2, eblk), lambda h, eb: (eb, 0, 0),
                         memory_space=pltpu.SMEM),
        ],
        out_specs=pl.BlockSpec(memory_space=pl.ANY),
        out_shape=jax.ShapeDtypeStruct((H, N, C), jnp.float32),
        scratch_shapes=[
            pltpu.VMEM((N, C), jnp.float32),
            pltpu.VMEM((N, C), jnp.float32),
            pltpu.VMEM((N, C + 128), jnp.float32),
            pltpu.SemaphoreType.DMA,
        ],
    )(xl_all, xr_all, aa, edges_r)

    out = pl.pallas_call(
        functools.partial(_head_kernel, nsub=nsub),
        grid=(N // nblk,),
        in_specs=[
            pl.BlockSpec((H, nblk, C), lambda i: (0, i, 0)),
            pl.BlockSpec((1, C), lambda i: (0, 0)),
            pl.BlockSpec((3, nsub), lambda i: (0, 0)),
            pl.BlockSpec((3, 1), lambda i: (0, 0), memory_space=pltpu.SMEM),
            pl.BlockSpec((C, outd), lambda i: (0, 0)),
            pl.BlockSpec((1, outd), lambda i: (0, 0)),
        ],
        out_specs=pl.BlockSpec((nblk, outd), lambda i: (i, 0)),
        out_shape=jax.ShapeDtypeStruct((N, outd), jnp.float32),
    )(agg, gat_bias.reshape(1, C), att_vec, att_bias, cls_W,
      cls_b.reshape(1, outd))
    return out


# speedup vs baseline: 11.8959x; 1.1407x over previous
import functools
import jax
import jax.numpy as jnp
from jax.experimental import pallas as pl
from jax.experimental.pallas import tpu as pltpu


def _pick_block(total, cap):
    best = 1
    for b in range(1, cap + 1):
        if total % b == 0:
            best = b
    return best


def _mm_kernel(x_ref, wl_ref, wr_ref, xl_ref, xr_ref):
    xl_ref[0] = jnp.dot(x_ref[...], wl_ref[0],
                        preferred_element_type=jnp.float32)
    xr_ref[0] = jnp.dot(x_ref[...], wr_ref[0],
                        preferred_element_type=jnp.float32)


def _edge_kernel(xl_hbm, xr_hbm, aa_ref, edges_ref, agg_hbm,
                 xl_s, xr_s, agg2_s, sem, *, eb_count, eblk, unroll, nc):
    h = pl.program_id(0)
    eb = pl.program_id(1)

    @pl.when(eb == 0)
    def _init():
        cp = pltpu.make_async_copy(xl_hbm.at[h], xl_s, sem)
        cp.start()
        cp.wait()
        cp = pltpu.make_async_copy(xr_hbm.at[h], xr_s, sem)
        cp.start()
        cp.wait()
        agg2_s[...] = jnp.zeros_like(agg2_s)

    aa = aa_ref[0]

    def body(ib, carry):
        base = ib * unroll
        msgs = []
        dsts = []
        for j in range(unroll):
            src = edges_ref[0, 0, base + j]
            dst = edges_ref[0, 1, base + j]
            xlrow = xl_s[pl.ds(src, 1), :]
            xrrow = xr_s[pl.ds(dst, 1), :]
            m = xlrow + xrrow
            e = jnp.where(m > 0, m, 0.2 * m)
            logit = jnp.sum(e * aa, axis=1, keepdims=True)
            ev = jnp.exp(logit)
            msgs.append(jnp.concatenate(
                [ev * xlrow, jnp.broadcast_to(ev, (1, 128))], axis=1))
            dsts.append(dst)
        for j in range(unroll):
            agg2_s[pl.ds(dsts[j], 1), :] += msgs[j]
        return carry

    jax.lax.fori_loop(0, eblk // unroll, body, 0)

    @pl.when(eb == eb_count - 1)
    def _fin():
        xl_s[...] = agg2_s[:, :nc] / (agg2_s[:, nc:nc + 1] + 1e-16)
        cp = pltpu.make_async_copy(xl_s, agg_hbm.at[h], sem)
        cp.start()
        cp.wait()


def _head_kernel(agg_ref, gb_ref, av_ref, ab_ref, cw_ref, cb_ref, out_ref,
                 *, nsub):
    z = jnp.mean(agg_ref[...], axis=0) + gb_ref[...]
    z = jnp.maximum(z, 0.0)
    parts = [z[:, k * nsub:(k + 1) * nsub] for k in range(3)]
    scs = []
    for k in range(3):
        s = jnp.sum(parts[k] * av_ref[k:k + 1, :], axis=1, keepdims=True)
        s = s + ab_ref[k, 0]
        scs.append(jnp.where(s > 0, s, 0.01 * s))
    scores = jnp.concatenate(scs, axis=1)
    mx = jnp.max(scores, axis=1, keepdims=True)
    ex = jnp.exp(scores - mx)
    alpha = ex / jnp.sum(ex, axis=1, keepdims=True)
    zw = jnp.concatenate([parts[i] * alpha[:, i:i + 1] for i in range(3)],
                         axis=1)
    out_ref[...] = jnp.dot(zw, cw_ref[...],
                           preferred_element_type=jnp.float32) + cb_ref[...]


def kernel(x, edge_index, W_l, W_r, att_a, gat_bias, att_vec, att_bias,
           cls_W, cls_b):
    N, D = x.shape
    E = edge_index.shape[1]
    H, C = att_a.shape
    nsub = att_vec.shape[1]
    outd = cls_W.shape[1]

    eblk = _pick_block(E, 2048)
    for b in range(32, 2049, 32):
        if E % b == 0:
            eblk = b
    ebc = E // eblk
    edges_r = edge_index.reshape(2, ebc, eblk).transpose(1, 0, 2)
    wl = W_l.reshape(D, H, C).transpose(1, 0, 2)
    wr = W_r.reshape(D, H, C).transpose(1, 0, 2)
    aa = att_a.reshape(H, 1, C)

    nblk = 1
    for b in range(8, 2049, 8):
        if N % b == 0:
            nblk = b

    xl_all, xr_all = pl.pallas_call(
        _mm_kernel,
        grid=(H, N // nblk),
        in_specs=[
            pl.BlockSpec((nblk, D), lambda h, i: (i, 0)),
            pl.BlockSpec((1, D, C), lambda h, i: (h, 0, 0)),
            pl.BlockSpec((1, D, C), lambda h, i: (h, 0, 0)),
        ],
        out_specs=[
            pl.BlockSpec((1, nblk, C), lambda h, i: (h, i, 0)),
            pl.BlockSpec((1, nblk, C), lambda h, i: (h, i, 0)),
        ],
        out_shape=[
            jax.ShapeDtypeStruct((H, N, C), jnp.float32),
            jax.ShapeDtypeStruct((H, N, C), jnp.float32),
        ],
    )(x, wl, wr)

    agg = pl.pallas_call(
        functools.partial(_edge_kernel, eb_count=ebc, eblk=eblk,
                          unroll=max(u for u in (1, 2, 4, 8, 16, 32, 64)
                                     if eblk % u == 0), nc=C),
        grid=(H, ebc),
        in_specs=[
            pl.BlockSpec(memory_space=pl.ANY),
            pl.BlockSpec(memory_space=pl.ANY),
            pl.BlockSpec((1, 1, C), lambda h, eb: (h, 0, 0)),
            pl.BlockSpec((1, 2, eblk), lambda h, eb: (eb, 0, 0),
                         memory_space=pltpu.SMEM),
        ],
        out_specs=pl.BlockSpec(memory_space=pl.ANY),
        out_shape=jax.ShapeDtypeStruct((H, N, C), jnp.float32),
        scratch_shapes=[
            pltpu.VMEM((N, C), jnp.float32),
            pltpu.VMEM((N, C), jnp.float32),
            pltpu.VMEM((N, C + 128), jnp.float32),
            pltpu.SemaphoreType.DMA,
        ],
    )(xl_all, xr_all, aa, edges_r)

    out = pl.pallas_call(
        functools.partial(_head_kernel, nsub=nsub),
        grid=(N // nblk,),
        in_specs=[
            pl.BlockSpec((H, nblk, C), lambda i: (0, i, 0)),
            pl.BlockSpec((1, C), lambda i: (0, 0)),
            pl.BlockSpec((3, nsub), lambda i: (0, 0)),
            pl.BlockSpec((3, 1), lambda i: (0, 0), memory_space=pltpu.SMEM),
            pl.BlockSpec((C, outd), lambda i: (0, 0)),
            pl.BlockSpec((1, outd), lambda i: (0, 0)),
        ],
        out_specs=pl.BlockSpec((nblk, outd), lambda i: (i, 0)),
        out_shape=jax.ShapeDtypeStruct((N, outd), jnp.float32),
    )(agg, gat_bias.reshape(1, C), att_vec, att_bias, cls_W,
      cls_b.reshape(1, outd))
    return out


# speedup vs baseline: 12.7539x; 1.0721x over previous
import functools
import jax
import jax.numpy as jnp
from jax.experimental import pallas as pl
from jax.experimental.pallas import tpu as pltpu


def _pick_block(total, cap):
    best = 1
    for b in range(1, cap + 1):
        if total % b == 0:
            best = b
    return best


def _mm_kernel(x_ref, wl_ref, wr_ref, xl_ref, xr_ref):
    xl_ref[0] = jnp.dot(x_ref[...], wl_ref[0],
                        preferred_element_type=jnp.float32)
    xr_ref[0] = jnp.dot(x_ref[...], wr_ref[0],
                        preferred_element_type=jnp.float32)


def _edge_kernel(xl_hbm, xr_hbm, aa_ref, edges_ref, agg_hbm,
                 xl_s, xr_s, agg2_s, sem, *, eb_count, eblk, unroll, nc):
    h = pl.program_id(0)
    eb = pl.program_id(1)

    @pl.when(eb == 0)
    def _init():
        cp = pltpu.make_async_copy(xl_hbm.at[h], xl_s, sem)
        cp.start()
        cp.wait()
        cp = pltpu.make_async_copy(xr_hbm.at[h], xr_s, sem)
        cp.start()
        cp.wait()
        agg2_s[...] = jnp.zeros_like(agg2_s)

    aa = aa_ref[0]

    def body(ib, carry):
        base = ib * unroll
        msgs = []
        dsts = []
        for j in range(unroll):
            src = edges_ref[0, 0, base + j]
            dst = edges_ref[0, 1, base + j]
            xlrow = xl_s[pl.ds(src, 1), :]
            xrrow = xr_s[pl.ds(dst, 1), :]
            m = xlrow + xrrow
            e = jnp.where(m > 0, m, 0.2 * m)
            logit = jnp.sum(e * aa, axis=1, keepdims=True)
            ev = jnp.exp(logit)
            msgs.append(jnp.concatenate(
                [ev * xlrow, jnp.broadcast_to(ev, (1, 128))], axis=1))
            dsts.append(dst)
        for j in range(unroll):
            agg2_s[pl.ds(dsts[j], 1), :] += msgs[j]
        return carry

    jax.lax.fori_loop(0, eblk // unroll, body, 0)

    @pl.when(eb == eb_count - 1)
    def _fin():
        xl_s[...] = agg2_s[:, :nc] / (agg2_s[:, nc:nc + 1] + 1e-16)
        cp = pltpu.make_async_copy(xl_s, agg_hbm.at[h], sem)
        cp.start()
        cp.wait()


def _head_kernel(agg_ref, gb_ref, av_ref, ab_ref, cw_ref, cb_ref, out_ref,
                 *, nsub):
    z = jnp.mean(agg_ref[...], axis=0) + gb_ref[...]
    z = jnp.maximum(z, 0.0)
    parts = [z[:, k * nsub:(k + 1) * nsub] for k in range(3)]
    scs = []
    for k in range(3):
        s = jnp.sum(parts[k] * av_ref[k:k + 1, :], axis=1, keepdims=True)
        s = s + ab_ref[k, 0]
        scs.append(jnp.where(s > 0, s, 0.01 * s))
    scores = jnp.concatenate(scs, axis=1)
    mx = jnp.max(scores, axis=1, keepdims=True)
    ex = jnp.exp(scores - mx)
    alpha = ex / jnp.sum(ex, axis=1, keepdims=True)
    zw = jnp.concatenate([parts[i] * alpha[:, i:i + 1] for i in range(3)],
                         axis=1)
    out_ref[...] = jnp.dot(zw, cw_ref[...],
                           preferred_element_type=jnp.float32) + cb_ref[...]


def kernel(x, edge_index, W_l, W_r, att_a, gat_bias, att_vec, att_bias,
           cls_W, cls_b):
    N, D = x.shape
    E = edge_index.shape[1]
    H, C = att_a.shape
    nsub = att_vec.shape[1]
    outd = cls_W.shape[1]

    eblk = _pick_block(E, 2048)
    for b in range(32, 2049, 32):
        if E % b == 0:
            eblk = b
    ebc = E // eblk
    edges_r = edge_index.reshape(2, ebc, eblk).transpose(1, 0, 2)
    wl = W_l.reshape(D, H, C).transpose(1, 0, 2)
    wr = W_r.reshape(D, H, C).transpose(1, 0, 2)
    aa = att_a.reshape(H, 1, C)

    nblk = 1
    for b in range(8, 2049, 8):
        if N % b == 0:
            nblk = b

    xl_all, xr_all = pl.pallas_call(
        _mm_kernel,
        grid=(H, N // nblk),
        in_specs=[
            pl.BlockSpec((nblk, D), lambda h, i: (i, 0)),
            pl.BlockSpec((1, D, C), lambda h, i: (h, 0, 0)),
            pl.BlockSpec((1, D, C), lambda h, i: (h, 0, 0)),
        ],
        out_specs=[
            pl.BlockSpec((1, nblk, C), lambda h, i: (h, i, 0)),
            pl.BlockSpec((1, nblk, C), lambda h, i: (h, i, 0)),
        ],
        out_shape=[
            jax.ShapeDtypeStruct((H, N, C), jnp.float32),
            jax.ShapeDtypeStruct((H, N, C), jnp.float32),
        ],
    )(x, wl, wr)

    agg = pl.pallas_call(
        functools.partial(_edge_kernel, eb_count=ebc, eblk=eblk,
                          unroll=max(u for u in (1, 2, 4, 8, 16, 32, 64, 100)
                                     if eblk % u == 0), nc=C),
        grid=(H, ebc),
        in_specs=[
            pl.BlockSpec(memory_space=pl.ANY),
            pl.BlockSpec(memory_space=pl.ANY),
            pl.BlockSpec((1, 1, C), lambda h, eb: (h, 0, 0)),
            pl.BlockSpec((1, 2, eblk), lambda h, eb: (eb, 0, 0),
                         memory_space=pltpu.SMEM),
        ],
        out_specs=pl.BlockSpec(memory_space=pl.ANY),
        out_shape=jax.ShapeDtypeStruct((H, N, C), jnp.float32),
        scratch_shapes=[
            pltpu.VMEM((N, C), jnp.float32),
            pltpu.VMEM((N, C), jnp.float32),
            pltpu.VMEM((N, C + 128), jnp.float32),
            pltpu.SemaphoreType.DMA,
        ],
    )(xl_all, xr_all, aa, edges_r)

    out = pl.pallas_call(
        functools.partial(_head_kernel, nsub=nsub),
        grid=(N // nblk,),
        in_specs=[
            pl.BlockSpec((H, nblk, C), lambda i: (0, i, 0)),
            pl.BlockSpec((1, C), lambda i: (0, 0)),
            pl.BlockSpec((3, nsub), lambda i: (0, 0)),
            pl.BlockSpec((3, 1), lambda i: (0, 0), memory_space=pltpu.SMEM),
            pl.BlockSpec((C, outd), lambda i: (0, 0)),
            pl.BlockSpec((1, outd), lambda i: (0, 0)),
        ],
        out_specs=pl.BlockSpec((nblk, outd), lambda i: (i, 0)),
        out_shape=jax.ShapeDtypeStruct((N, outd), jnp.float32),
    )(agg, gat_bias.reshape(1, C), att_vec, att_bias, cls_W,
      cls_b.reshape(1, outd))
    return out
